# baseline (speedup 1.0000x reference)
import jax
import jax.numpy as jnp
import numpy as np
from jax import lax
from jax.experimental import pallas as pl
from jax.experimental.pallas import tpu as pltpu
from jax.experimental.pallas import tpu_sc as plsc

R = 128
C = 32768
L = 16
NV = C // L
S = 8
NVS = NV // S
NVS_BITS = NVS.bit_length() - 1
NBINS = 256
NC, NS = 2, 16
NW = NC * NS
ROWS_PER_W = R // NW

_MIN32 = np.int32(-2147483648)


def _encode(b):
    flip = (b >> 31) | _MIN32
    return b ^ flip


def _decode(k):
    flip = (~k >> 31) | _MIN32
    return k ^ flip


def _radix_pass(src, dst, hists, tot, sh, *, encode, decode, direct,
                pre_perm=None):
    lane = lax.iota(jnp.int32, L)
    ones = jnp.ones((L,), jnp.int32)
    zeros = jnp.zeros((L,), jnp.int32)

    def keys_at(j):
        k = plsc.bitcast(src[pl.ds(j * L, L)], jnp.int32)
        if encode:
            k = _encode(k)
        return k

    def digit_idx(k):
        d = lax.shift_right_logical(k, sh) & (NBINS - 1)
        return (d << 4) | lane

    @plsc.parallel_loop(0, NBINS, step=1)
    def _zero(t):
        for s in range(S):
            hists[s][pl.ds(t * L, L)] = zeros

    @plsc.parallel_loop(0, NVS, step=1)
    def _hist(t):
        for s in range(S):
            idx = digit_idx(keys_at(s * NVS + t))
            plsc.addupdate_scatter(hists[s], [idx], ones)

    @plsc.parallel_loop(0, NBINS, step=1)
    def _scan_local(t):
        pref = jnp.int32(0)
        for s in range(S):
            v = hists[s][pl.ds(t * L, L)]
            c = plsc.cumsum(v)
            hists[s][pl.ds(t * L, L)] = c - v + pref
            pref = pref + c[15]
        plsc.store_scatter(tot, [t + zeros], pref + zeros, mask=lane == 0)

    def scan_tot(u, carry):
        v = tot[pl.ds(u * L, L)]
        c = plsc.cumsum(v)
        tot[pl.ds(u * L, L)] = c - v + carry
        return carry + c[15]

    lax.fori_loop(0, NBINS // L, scan_tot, jnp.int32(0))

    @plsc.parallel_loop(0, NBINS, step=1)
    def _scan_fix(t):
        g = plsc.load_gather(tot, [t + zeros])
        for s in range(S):
            hists[s][pl.ds(t * L, L)] = hists[s][pl.ds(t * L, L)] + g

    def load_stage(t):
        ks = tuple(keys_at(s * NVS + t) for s in range(S))
        idxs = tuple(digit_idx(k) for k in ks)
        return ks, idxs

    def perm_body(t, carry):
        ks, idxs = carry
        nxt = load_stage(jnp.minimum(t + 1, NVS - 1))
        rs = [plsc.load_gather(hists[s], [idxs[s]]) for s in range(S)]
        for s in range(S):
            plsc.addupdate_scatter(hists[s], [idxs[s]], ones)
        for s in range(S):
            k = ks[s]
            r = rs[s]
            if decode:
                k = _decode(k)
            if direct:
                p = r
            else:
                p = ((r & (C - NVS * L))
                     | ((r & (NVS - 1)) << 4)
                     | (lax.shift_right_logical(r, NVS_BITS) & (L - 1)))
            plsc.store_scatter(dst, [p], plsc.bitcast(k, jnp.float32))
        return nxt

    if pre_perm is not None:
        pre_perm()
    lax.fori_loop(0, NVS, perm_body, load_stage(0))


def _sort_body(x_hbm, out_hbm, buf_a, buf_b, out_sem, *tables):
    wid = lax.axis_index("s") * NC + lax.axis_index("c")
    hists = list(tables[:S])
    tot = tables[S]

    def sort_row(row, bx, by, wait_prev):
        pltpu.sync_copy(x_hbm.at[row], bx)
        _radix_pass(bx, by, hists, tot, 0, encode=True, decode=False,
                    direct=False, pre_perm=wait_prev)
        _radix_pass(by, bx, hists, tot, 8, encode=False, decode=False,
                    direct=False)
        _radix_pass(bx, by, hists, tot, 16, encode=False, decode=False,
                    direct=False)
        _radix_pass(by, bx, hists, tot, 24, encode=False, decode=True,
                    direct=True)
        return pltpu.async_copy(bx, out_hbm.at[row], out_sem)

    def pair_body(t, c):
        row = wid * ROWS_PER_W + 2 * t

        def wait_prev_pair():
            @pl.when(t > 0)
            def _():
                pltpu.make_async_copy(buf_b, out_hbm.at[row - 1],
                                      out_sem).wait()

        h0 = sort_row(row, buf_a, buf_b, wait_prev_pair)
        sort_row(row + 1, buf_b, buf_a, h0.wait)
        return c

    lax.fori_loop(0, ROWS_PER_W // 2, pair_body, 0)
    last = wid * ROWS_PER_W + ROWS_PER_W - 1
    pltpu.make_async_copy(buf_b, out_hbm.at[last], out_sem).wait()


@jax.jit
def _sort(x):
    mesh = plsc.VectorSubcoreMesh(core_axis_name="c", subcore_axis_name="s",
                                  num_cores=NC, num_subcores=NS)
    return pl.kernel(
        _sort_body,
        out_type=jax.ShapeDtypeStruct((R, C), jnp.float32),
        mesh=mesh,
        compiler_params=pltpu.CompilerParams(needs_layout_passes=False),
        scratch_types=(
            [pltpu.VMEM((C,), jnp.float32),
             pltpu.VMEM((C,), jnp.float32),
             pltpu.SemaphoreType.DMA]
            + [pltpu.VMEM((NBINS * L,), jnp.int32) for _ in range(S)]
            + [pltpu.VMEM((NBINS,), jnp.int32)]
        ),
    )(x)


def kernel(inputs):
    return _sort(inputs)


# speedup vs baseline: 1.1526x; 1.1526x over previous
import jax
import jax.numpy as jnp
import numpy as np
from jax import lax
from jax.experimental import pallas as pl
from jax.experimental.pallas import tpu as pltpu
from jax.experimental.pallas import tpu_sc as plsc

R = 128
C = 32768
L = 16
NV = C // L
S = 8
NVS = NV // S
NVS_BITS = NVS.bit_length() - 1
NBINS = 256
NC, NS = 2, 16
NW = NC * NS
ROWS_PER_W = R // NW

_MIN32 = np.int32(-2147483648)


def _encode(b):
    flip = (b >> 31) | _MIN32
    return b ^ flip


def _decode(k):
    flip = (~k >> 31) | _MIN32
    return k ^ flip


def _radix_pass(src, dst, hists, tot, sh, *, encode, decode, direct,
                pre_perm=None):
    lane = lax.iota(jnp.int32, L)
    ones = jnp.ones((L,), jnp.int32)
    zeros = jnp.zeros((L,), jnp.int32)

    def keys_at(j):
        k = plsc.bitcast(src[pl.ds(j * L, L)], jnp.int32)
        if encode:
            k = _encode(k)
        return k

    def digit_idx(k):
        d = lax.shift_right_logical(k, sh) & (NBINS - 1)
        return (d << 4) | lane

    @plsc.parallel_loop(0, NBINS, step=1)
    def _zero(t):
        for s in range(S):
            hists[s][pl.ds(t * L, L)] = zeros

    @plsc.parallel_loop(0, NVS, step=1)
    def _hist(t):
        for s in range(S):
            idx = digit_idx(keys_at(s * NVS + t))
            plsc.addupdate_scatter(hists[s], [idx], ones)

    @plsc.parallel_loop(0, NBINS, step=1)
    def _scan_local(t):
        vs = [hists[s][pl.ds(t * L, L)] for s in range(S)]
        tcross = vs[0] + vs[1]
        for s in range(2, S):
            tcross = tcross + vs[s]
        cinc = plsc.cumsum(tcross)
        acc = cinc - tcross
        for s in range(S):
            hists[s][pl.ds(t * L, L)] = acc
            acc = acc + vs[s]
        plsc.store_scatter(tot, [t + zeros], cinc[15] + zeros,
                           mask=lane == 0)

    def scan_tot(u, carry):
        v = tot[pl.ds(u * L, L)]
        c = plsc.cumsum(v)
        tot[pl.ds(u * L, L)] = c - v + carry
        return carry + c[15]

    lax.fori_loop(0, NBINS // L, scan_tot, jnp.int32(0))

    @plsc.parallel_loop(0, NBINS, step=1)
    def _scan_fix(t):
        g = plsc.load_gather(tot, [t + zeros])
        for s in range(S):
            hists[s][pl.ds(t * L, L)] = hists[s][pl.ds(t * L, L)] + g

    def load_stage(t):
        ks = tuple(keys_at(s * NVS + t) for s in range(S))
        idxs = tuple(digit_idx(k) for k in ks)
        return ks, idxs

    def perm_body(t, carry):
        ks, idxs = carry
        nxt = load_stage(jnp.minimum(t + 1, NVS - 1))
        rs = [plsc.load_gather(hists[s], [idxs[s]]) for s in range(S)]
        for s in range(S):
            plsc.addupdate_scatter(hists[s], [idxs[s]], ones)
        for s in range(S):
            k = ks[s]
            r = rs[s]
            if decode:
                k = _decode(k)
            if direct:
                p = r
            else:
                p = (((r & (NV - 1)) << 4)
                     | lax.shift_right_logical(r, NV.bit_length() - 1))
            plsc.store_scatter(dst, [p], plsc.bitcast(k, jnp.float32))
        return nxt

    if pre_perm is not None:
        pre_perm()
    lax.fori_loop(0, NVS, perm_body, load_stage(0))


def _sort_body(x_hbm, out_hbm, buf_a, buf_b, *tables):
    wid = lax.axis_index("s") * NC + lax.axis_index("c")
    hists = list(tables[:S])
    tot = tables[S]

    def row_body(t, c):
        row = wid * ROWS_PER_W + t
        pltpu.sync_copy(x_hbm.at[row], buf_a)
        _radix_pass(buf_a, buf_b, hists, tot, 0, encode=True, decode=False,
                    direct=False)
        _radix_pass(buf_b, buf_a, hists, tot, 8, encode=False, decode=False,
                    direct=False)
        _radix_pass(buf_a, buf_b, hists, tot, 16, encode=False, decode=False,
                    direct=False)
        _radix_pass(buf_b, buf_a, hists, tot, 24, encode=False, decode=True,
                    direct=True)
        pltpu.sync_copy(buf_a, out_hbm.at[row])
        return c

    lax.fori_loop(0, ROWS_PER_W, row_body, 0)


@jax.jit
def _sort(x):
    mesh = plsc.VectorSubcoreMesh(core_axis_name="c", subcore_axis_name="s",
                                  num_cores=NC, num_subcores=NS)
    return pl.kernel(
        _sort_body,
        out_type=jax.ShapeDtypeStruct((R, C), jnp.float32),
        mesh=mesh,
        compiler_params=pltpu.CompilerParams(needs_layout_passes=False),
        scratch_types=(
            [pltpu.VMEM((C,), jnp.float32),
             pltpu.VMEM((C,), jnp.float32)]
            + [pltpu.VMEM((NBINS * L,), jnp.int32) for _ in range(S)]
            + [pltpu.VMEM((NBINS,), jnp.int32)]
        ),
    )(x)


def kernel(inputs):
    return _sort(inputs)


# speedup vs baseline: 1.1532x; 1.0005x over previous
import jax
import jax.numpy as jnp
import numpy as np
from jax import lax
from jax.experimental import pallas as pl
from jax.experimental.pallas import tpu as pltpu
from jax.experimental.pallas import tpu_sc as plsc

R = 128
C = 32768
L = 16
NV = C // L
S = 8
NVS = NV // S
NBINS = 256
NC, NS = 2, 16
NW = NC * NS
ROWS_PER_W = R // NW

_MIN32 = np.int32(-2147483648)


def _encode(b):
    flip = (b >> 31) | _MIN32
    return b ^ flip


def _decode(k):
    flip = (~k >> 31) | _MIN32
    return k ^ flip


def _radix_pass(src, dst, hists, tot, sh, *, encode, decode, direct):
    lane = lax.iota(jnp.int32, L)
    ones = jnp.ones((L,), jnp.int32)
    zeros = jnp.zeros((L,), jnp.int32)

    def keys_at(j):
        k = plsc.bitcast(src[pl.ds(j * L, L)], jnp.int32)
        if encode:
            k = _encode(k)
        return k

    def digit_idx(k):
        d = lax.shift_right_logical(k, sh) & (NBINS - 1)
        return (d << 4) | lane

    @plsc.parallel_loop(0, NBINS, step=1)
    def _zero(t):
        for s in range(S):
            hists[s][pl.ds(t * L, L)] = zeros

    @plsc.parallel_loop(0, NVS, step=1)
    def _hist(t):
        for s in range(S):
            idx = digit_idx(keys_at(s * NVS + t))
            plsc.addupdate_scatter(hists[s], [idx], ones)

    @plsc.parallel_loop(0, NBINS, step=1)
    def _scan_local(t):
        vs = [hists[s][pl.ds(t * L, L)] for s in range(S)]
        tcross = vs[0] + vs[1]
        for s in range(2, S):
            tcross = tcross + vs[s]
        cinc = plsc.cumsum(tcross)
        acc = cinc - tcross
        for s in range(S):
            hists[s][pl.ds(t * L, L)] = acc
            acc = acc + vs[s]
        plsc.store_scatter(tot, [t + zeros], cinc[15] + zeros,
                           mask=lane == 0)

    def scan_tot(u, carry):
        v = tot[pl.ds(u * L, L)]
        c = plsc.cumsum(v)
        tot[pl.ds(u * L, L)] = c - v + carry
        return carry + c[15]

    lax.fori_loop(0, NBINS // L, scan_tot, jnp.int32(0))

    @plsc.parallel_loop(0, NBINS, step=1)
    def _scan_fix(t):
        g = plsc.load_gather(tot, [t + zeros])
        for s in range(S):
            hists[s][pl.ds(t * L, L)] = hists[s][pl.ds(t * L, L)] + g

    def load_stage(t):
        ks = tuple(keys_at(s * NVS + t) for s in range(S))
        idxs = tuple(digit_idx(k) for k in ks)
        return ks, idxs

    def perm_body(t, carry):
        ks, idxs = carry
        nxt = load_stage(jnp.minimum(t + 1, NVS - 1))
        rs = [plsc.load_gather(hists[s], [idxs[s]]) for s in range(S)]
        for s in range(S):
            plsc.addupdate_scatter(hists[s], [idxs[s]], ones)
        for s in range(S):
            k = ks[s]
            r = rs[s]
            if decode:
                k = _decode(k)
            if direct:
                p = r
            else:
                p = (((r & (NV - 1)) << 4)
                     | lax.shift_right_logical(r, NV.bit_length() - 1))
            plsc.store_scatter(dst, [p], plsc.bitcast(k, jnp.float32))
        return nxt

    lax.fori_loop(0, NVS, perm_body, load_stage(0))


def _sort_body(x_hbm, out_hbm, buf_a, buf_b, *tables):
    wid = lax.axis_index("s") * NC + lax.axis_index("c")
    hists = list(tables[:S])
    tot = tables[S]

    def row_body(t, c):
        row = wid * ROWS_PER_W + t
        pltpu.sync_copy(x_hbm.at[row], buf_a)
        _radix_pass(buf_a, buf_b, hists, tot, 0, encode=True, decode=False,
                    direct=False)
        _radix_pass(buf_b, buf_a, hists, tot, 8, encode=False, decode=False,
                    direct=False)
        _radix_pass(buf_a, buf_b, hists, tot, 16, encode=False, decode=False,
                    direct=False)
        _radix_pass(buf_b, buf_a, hists, tot, 24, encode=False, decode=True,
                    direct=True)
        pltpu.sync_copy(buf_a, out_hbm.at[row])
        return c

    lax.fori_loop(0, ROWS_PER_W, row_body, 0)


@jax.jit
def _sort(x):
    mesh = plsc.VectorSubcoreMesh(core_axis_name="c", subcore_axis_name="s",
                                  num_cores=NC, num_subcores=NS)
    return pl.kernel(
        _sort_body,
        out_type=jax.ShapeDtypeStruct((R, C), jnp.float32),
        mesh=mesh,
        compiler_params=pltpu.CompilerParams(needs_layout_passes=False),
        scratch_types=(
            [pltpu.VMEM((C,), jnp.float32),
             pltpu.VMEM((C,), jnp.float32)]
            + [pltpu.VMEM((NBINS * L,), jnp.int32) for _ in range(S)]
            + [pltpu.VMEM((NBINS,), jnp.int32)]
        ),
    )(x)


def kernel(inputs):
    return _sort(inputs)
